# baseline (speedup 1.0000x reference)
import functools

import jax
import jax.numpy as jnp
from jax import lax
from jax.experimental import pallas as pl
from jax.experimental.pallas import tpu as pltpu
from jax.experimental.pallas import tpu_sc as plsc

F32 = jnp.float32
I32 = jnp.int32

N = 10000
E = 320000
D_IN = 128
H = 8
F = 64
D_MID = H * F

NC = 2
NS = 16
NWORK = NC * NS

NP = 10240
RPT = NP // NS
EPT = E // NWORK
W = 80
NWIN = EPT // W

RB = 1024
NB = NP // RB

_MESH = dict(core_axis_name="c", subcore_axis_name="s")

_GDN = jax.lax.GatherDimensionNumbers(
    offset_dims=(), collapsed_slice_dims=(0,), start_index_map=(0,))


def _permute16(v, idx):
    return jax.lax.gather(
        v, idx[:, None], _GDN, slice_sizes=(1,),
        mode=jax.lax.GatherScatterMode.PROMISE_IN_BOUNDS)


HP = H // 2
FP = 2 * F


def _proj_body(x_ref, wl_ref, bl_ref, wr_ref, br_ref, xl_ref, xr_ref):
    xb = x_ref[...]
    xl = jnp.dot(xb, wl_ref[...], preferred_element_type=F32) + bl_ref[...]
    xr = jnp.dot(xb, wr_ref[...], preferred_element_type=F32) + br_ref[...]
    for h in range(H):
        xl_ref[h] = xl[:, h * F:(h + 1) * F]
        xr_ref[h] = xr[:, h * F:(h + 1) * F]


def _proj(xp, Wl1, bl1r, Wr1, br1r):
    return pl.pallas_call(
        _proj_body,
        grid=(NB,),
        in_specs=[
            pl.BlockSpec((RB, D_IN), lambda i: (i, 0)),
            pl.BlockSpec((D_IN, D_MID), lambda i: (0, 0)),
            pl.BlockSpec((1, D_MID), lambda i: (0, 0)),
            pl.BlockSpec((D_IN, D_MID), lambda i: (0, 0)),
            pl.BlockSpec((1, D_MID), lambda i: (0, 0)),
        ],
        out_specs=[
            pl.BlockSpec((H, RB, F), lambda i: (0, i, 0)),
            pl.BlockSpec((H, RB, F), lambda i: (0, i, 0)),
        ],
        out_shape=[
            jax.ShapeDtypeStruct((H, NP, F), F32),
            jax.ShapeDtypeStruct((H, NP, F), F32),
        ],
    )(xp, Wl1, bl1r, Wr1, br1r)


NBUF = 3
EPT2 = E // NS
NWIN2 = EPT2 // W


def _edge1_body(xl_hbm, xr_hbm, att_hbm, src_hbm, dst_hbm,
                acc_out, den_out,
                shacc, shden, att_v,
                sidx_v, didx_v, didxg_v, a_v, b_v, ex_v, t_v, isem, gsem):
    c = lax.axis_index("c")
    s = lax.axis_index("s")
    base = s * EPT2
    row0 = s * RPT
    cnp = c * NP

    z16 = jnp.zeros((16,), F32)

    def _zrow(r, _):
        for k in range(F // 16):
            a_v[0, r, pl.ds(k * 16, 16)] = z16
        return _

    lax.fori_loop(0, W, _zrow, None)

    def _zden(r, _):
        ex_v[0, pl.ds(r * 16, 16)] = z16
        return _

    lax.fori_loop(0, W // 16, _zden, None)

    for q in range(RPT // W):
        pltpu.sync_copy(a_v.at[0], shacc.at[pl.ds(row0 + q * W, W)])
        pltpu.sync_copy(ex_v.at[0], shden.at[pl.ds(row0 + q * W, W)])
    pltpu.sync_copy(att_hbm, att_v)
    plsc.subcore_barrier()

    nk = F // 16
    attv = [att_v[pl.ds(c * F + k * 16, 16)] for k in range(nk)]
    iota16 = jax.lax.iota(I32, 16)

    def _lidx(w, q):
        e0 = base + w * W
        pltpu.async_copy(src_hbm.at[pl.ds(e0, W)], sidx_v.at[q], isem.at[q])
        pltpu.async_copy(dst_hbm.at[pl.ds(e0, W)], didx_v.at[q], isem.at[q])

    def _didx(q):
        pltpu.make_async_copy(
            src_hbm.at[pl.ds(0, W)], sidx_v.at[q], isem.at[q]).wait()
        pltpu.make_async_copy(
            src_hbm.at[pl.ds(0, W)], didx_v.at[q], isem.at[q]).wait()

    def _shift(q):
        for t in range(W // 16):
            sl = pl.ds(t * 16, 16)
            sidx_v[q, sl] = sidx_v[q, sl] + cnp
            didxg_v[q, sl] = didx_v[q, sl] + cnp

    def _gissue(q):
        pltpu.async_copy(xl_hbm.at[sidx_v.at[q]], a_v.at[q], gsem.at[q])
        pltpu.async_copy(xr_hbm.at[didxg_v.at[q]], b_v.at[q], gsem.at[q])

    def _gdrain(q):
        pltpu.make_async_copy(
            xl_hbm.at[sidx_v.at[q]], a_v.at[q], gsem.at[q]).wait()
        pltpu.make_async_copy(
            xr_hbm.at[didxg_v.at[q]], b_v.at[q], gsem.at[q]).wait()

    def _compute(q):
        def _group(g, _):
            r0 = g * 16
            for e in range(16):
                sv = None
                for k in range(nk):
                    av = a_v[q, r0 + e, pl.ds(k * 16, 16)]
                    bv = b_v[q, r0 + e, pl.ds(k * 16, 16)]
                    z = av + bv
                    m = jnp.maximum(z, 0.2 * z)
                    tk = m * attv[k]
                    sv = tk if sv is None else sv + tk
                t_v[e, pl.ds(0, 16)] = sv
            alpha = None
            for l in range(16):
                col = plsc.load_gather(t_v, [iota16, jnp.full((16,), l, I32)])
                alpha = col if alpha is None else alpha + col
            ex = jnp.exp(alpha)
            ex_v[q, pl.ds(r0, 16)] = ex
            for e in range(16):
                es = ex[e]
                for k in range(nk):
                    a_v[q, r0 + e, pl.ds(k * 16, 16)] = (
                        a_v[q, r0 + e, pl.ds(k * 16, 16)] * es)
            return _

        lax.fori_loop(0, W // 16, _group, None)

    def _scatter(q):
        pltpu.sync_copy(ex_v.at[q], shden.at[didx_v.at[q]], add=True)
        pltpu.sync_copy(a_v.at[q], shacc.at[didx_v.at[q]], add=True)

    _lidx(0, 0)
    _lidx(1, 1)
    _didx(0)
    _shift(0)
    _gissue(0)
    _lidx(2, 2)

    def _step(w, carry):
        q = lax.rem(w, NBUF)
        q1 = lax.rem(w + 1, NBUF)

        @pl.when(w + 1 < NWIN2)
        def _():
            _didx(q1)
            _shift(q1)
            _gissue(q1)

        _gdrain(q)
        _compute(q)
        _scatter(q)

        @pl.when(w + NBUF < NWIN2)
        def _():
            _lidx(w + NBUF, q)

        return carry

    lax.fori_loop(0, NWIN2, _step, 0)

    plsc.subcore_barrier()
    out0 = cnp + row0
    pltpu.sync_copy(shacc.at[pl.ds(row0, RPT)], acc_out.at[pl.ds(out0, RPT)])
    pltpu.sync_copy(shden.at[pl.ds(row0, RPT)], den_out.at[pl.ds(out0, RPT)])


def _edge1(xl_p, xr_p, att_p, src, dst):
    k = functools.partial(
        pl.kernel,
        out_type=(
            jax.ShapeDtypeStruct((NC * NP, F), F32),
            jax.ShapeDtypeStruct((NC * NP,), F32),
        ),
        mesh=plsc.VectorSubcoreMesh(**_MESH),
        compiler_params=pltpu.CompilerParams(needs_layout_passes=False, use_tc_tiling_on_sc=False),
        scratch_types=[
            pltpu.VMEM_SHARED((NP, F), F32),
            pltpu.VMEM_SHARED((NP,), F32),
            pltpu.VMEM((2 * F,), F32),
            pltpu.VMEM((NBUF, W), I32),
            pltpu.VMEM((NBUF, W), I32),
            pltpu.VMEM((NBUF, W), I32),
            pltpu.VMEM((NBUF, W, F), F32),
            pltpu.VMEM((NBUF, W, F), F32),
            pltpu.VMEM((NBUF, W), F32),
            pltpu.VMEM((16, 16), F32),
            pltpu.SemaphoreType.DMA((NBUF,)),
            pltpu.SemaphoreType.DMA((NBUF,)),
        ],
    )(_edge1_body)
    return k(xl_p, xr_p, att_p, src, dst)


def _combine_body(acc_ref, den_ref, bias_ref, wl2_ref, wr2_ref, sc_ref,
                  lr2_ref):
    rden = 1.0 / (den_ref[...] + 1e-16)
    xl2 = None
    xr2 = None
    for h in range(H):
        piece = acc_ref[h] * rden[h][:, None] + bias_ref[h][None, :]
        piece = jnp.maximum(piece, 0.0)
        cl = jnp.sum(piece * wl2_ref[h][None, :], axis=1)
        cr = jnp.sum(piece * wr2_ref[h][None, :], axis=1)
        xl2 = cl if xl2 is None else xl2 + cl
        xr2 = cr if xr2 is None else xr2 + cr
    lr2_ref[0, :] = xl2 + sc_ref[0, 0]
    lr2_ref[1, :] = xr2 + sc_ref[0, 1]


def _combine(accs, dens, bias1hf, wl2hf, wr2hf, sc2):
    return pl.pallas_call(
        _combine_body,
        grid=(NB,),
        in_specs=[
            pl.BlockSpec((H, RB, F), lambda i: (0, i, 0)),
            pl.BlockSpec((H, RB), lambda i: (0, i)),
            pl.BlockSpec((H, F), lambda i: (0, 0)),
            pl.BlockSpec((H, F), lambda i: (0, 0)),
            pl.BlockSpec((H, F), lambda i: (0, 0)),
            pl.BlockSpec((1, 8), lambda i: (0, 0)),
        ],
        out_specs=pl.BlockSpec((NC, RB), lambda i: (0, i)),
        out_shape=jax.ShapeDtypeStruct((NC, NP), F32),
    )(accs, dens, bias1hf, wl2hf, wr2hf, sc2)


def _edge2_body(xl2_hbm, xr2_hbm, att_hbm, src_hbm, dst_hbm,
                num_out, den_out, ex_out,
                shnum, shden, xl2_v, xr2_v, att_v, sidx_v, didx_v,
                ex_v, nm_v, zden_v):
    c = lax.axis_index("c")
    s = lax.axis_index("s")
    wid = c * NS + s
    base = wid * EPT
    row0 = s * RPT

    z16 = jnp.zeros((16,), F32)

    def _zden(r, _):
        zden_v[pl.ds(r * 16, 16)] = z16
        return _

    lax.fori_loop(0, RPT // 16, _zden, None)
    pltpu.sync_copy(zden_v, shnum.at[pl.ds(row0, RPT)])
    pltpu.sync_copy(zden_v, shden.at[pl.ds(row0, RPT)])
    pltpu.sync_copy(xl2_hbm, xl2_v)
    pltpu.sync_copy(xr2_hbm, xr2_v)
    pltpu.sync_copy(att_hbm, att_v)
    plsc.subcore_barrier()

    atts = att_v[pl.ds(0, 16)][0]

    def _window(w, _):
        e0 = base + w * W
        pltpu.sync_copy(src_hbm.at[pl.ds(e0, W)], sidx_v)
        pltpu.sync_copy(dst_hbm.at[pl.ds(e0, W)], didx_v)

        def _group(g, _):
            r0 = g * 16
            sv = sidx_v[pl.ds(r0, 16)]
            dv = didx_v[pl.ds(r0, 16)]
            xls = plsc.load_gather(xl2_v, [sv])
            xrd = plsc.load_gather(xr2_v, [dv])
            z = xls + xrd
            m = jnp.maximum(z, 0.2 * z)
            ex = jnp.exp(atts * m)
            ex_v[pl.ds(r0, 16)] = ex
            nm_v[pl.ds(r0, 16)] = ex * xls
            return _

        lax.fori_loop(0, W // 16, _group, None)

        pltpu.sync_copy(ex_v, ex_out.at[pl.ds(e0, W)])
        pltpu.sync_copy(ex_v, shden.at[didx_v], add=True)
        pltpu.sync_copy(nm_v, shnum.at[didx_v], add=True)
        return _

    lax.fori_loop(0, NWIN, _window, None)
    plsc.subcore_barrier()

    out0 = c * NP + row0
    pltpu.sync_copy(shnum.at[pl.ds(row0, RPT)], num_out.at[pl.ds(out0, RPT)])
    pltpu.sync_copy(shden.at[pl.ds(row0, RPT)], den_out.at[pl.ds(out0, RPT)])


def _edge2(xl2, xr2, att2p, src, dst):
    k = functools.partial(
        pl.kernel,
        out_type=(
            jax.ShapeDtypeStruct((NC * NP,), F32),
            jax.ShapeDtypeStruct((NC * NP,), F32),
            jax.ShapeDtypeStruct((E,), F32),
        ),
        mesh=plsc.VectorSubcoreMesh(**_MESH),
        compiler_params=pltpu.CompilerParams(needs_layout_passes=False, use_tc_tiling_on_sc=False),
        scratch_types=[
            pltpu.VMEM_SHARED((NP,), F32),
            pltpu.VMEM_SHARED((NP,), F32),
            pltpu.VMEM((NP,), F32),
            pltpu.VMEM((NP,), F32),
            pltpu.VMEM((16,), F32),
            pltpu.VMEM((W,), I32),
            pltpu.VMEM((W,), I32),
            pltpu.VMEM((W,), F32),
            pltpu.VMEM((W,), F32),
            pltpu.VMEM((RPT,), F32),
        ],
    )(_edge2_body)
    return k(xl2, xr2, att2p, src, dst)


def _final2_body(num_ref, den_ref, sc_ref, h2_ref, rden_ref):
    dent = den_ref[0] + den_ref[1]
    numt = num_ref[0] + num_ref[1]
    r = 1.0 / (dent + 1e-16)
    h2_ref[0, :] = numt * r + sc_ref[0, 2]
    rden_ref[0, :] = r


def _final2(num2, den2, sc2):
    return pl.pallas_call(
        _final2_body,
        grid=(NB,),
        in_specs=[
            pl.BlockSpec((NC, RB), lambda i: (0, i)),
            pl.BlockSpec((NC, RB), lambda i: (0, i)),
            pl.BlockSpec((1, 8), lambda i: (0, 0)),
        ],
        out_specs=[
            pl.BlockSpec((1, RB), lambda i: (0, i)),
            pl.BlockSpec((1, RB), lambda i: (0, i)),
        ],
        out_shape=[
            jax.ShapeDtypeStruct((1, NP), F32),
            jax.ShapeDtypeStruct((1, NP), F32),
        ],
    )(num2, den2, sc2)


def _edge2b_body(rden_hbm, ex_hbm, dst_hbm, a2_out,
                 rden_v, didx_v, ex_v, a_v):
    c = lax.axis_index("c")
    s = lax.axis_index("s")
    base = (c * NS + s) * EPT
    pltpu.sync_copy(rden_hbm, rden_v)

    def _window(w, _):
        e0 = base + w * W
        pltpu.sync_copy(dst_hbm.at[pl.ds(e0, W)], didx_v)
        pltpu.sync_copy(ex_hbm.at[pl.ds(e0, W)], ex_v)

        def _group(g, _):
            r0 = g * 16
            dv = didx_v[pl.ds(r0, 16)]
            r = plsc.load_gather(rden_v, [dv])
            a_v[pl.ds(r0, 16)] = ex_v[pl.ds(r0, 16)] * r
            return _

        lax.fori_loop(0, W // 16, _group, None)
        pltpu.sync_copy(a_v, a2_out.at[pl.ds(e0, W)])
        return _

    lax.fori_loop(0, NWIN, _window, None)


def _edge2b(rden, ex2, dst):
    k = functools.partial(
        pl.kernel,
        out_type=jax.ShapeDtypeStruct((E,), F32),
        mesh=plsc.VectorSubcoreMesh(**_MESH),
        compiler_params=pltpu.CompilerParams(needs_layout_passes=False, use_tc_tiling_on_sc=False),
        scratch_types=[
            pltpu.VMEM((NP,), F32),
            pltpu.VMEM((W,), I32),
            pltpu.VMEM((W,), F32),
            pltpu.VMEM((W,), F32),
        ],
    )(_edge2b_body)
    return k(rden, ex2, dst)


def kernel(x, edge_index, Wl1, bl1, Wr1, br1, att1, bias1,
           Wl2, bl2, Wr2, br2, att2, bias2):
    xp = jnp.pad(x, ((0, NP - N), (0, 0)))
    src = edge_index[0]
    dst = edge_index[1]

    XL, XR = _proj(xp, Wl1, bl1.reshape(1, D_MID), Wr1, br1.reshape(1, D_MID))

    XLp = XL.reshape(HP, NC * NP, F)
    XRp = XR.reshape(HP, NC * NP, F)
    att1p = att1.reshape(HP, 2 * F)
    accs = []
    dens = []
    for p in range(HP):
        acc_p, den_p = _edge1(XLp[p], XRp[p], att1p[p], src, dst)
        accs.append(acc_p.reshape(NC, NP, F))
        dens.append(den_p.reshape(NC, NP))
    accs = jnp.concatenate(accs)
    dens = jnp.concatenate(dens)

    sc2 = jnp.stack([bl2[0], br2[0], bias2[0], att2[0, 0],
                     0.0, 0.0, 0.0, 0.0]).reshape(1, 8).astype(F32)
    att2p = jnp.pad(att2.reshape(-1), (0, 15)).astype(F32)

    lr2 = _combine(accs, dens, bias1.reshape(H, F),
                   Wl2.reshape(H, F), Wr2.reshape(H, F), sc2)

    num2f, den2f, ex2 = _edge2(lr2[0], lr2[1], att2p, src, dst)
    num2 = num2f.reshape(NC, NP)
    den2 = den2f.reshape(NC, NP)

    h2row, rdenrow = _final2(num2, den2, sc2)
    a2 = _edge2b(rdenrow.reshape(NP), ex2, dst)

    h2 = h2row[0, :N].reshape(N, 1)
    return (h2, edge_index, a2.reshape(E, 1))


# speedup vs baseline: 1.5101x; 1.5101x over previous
import functools

import jax
import jax.numpy as jnp
from jax import lax
from jax.experimental import pallas as pl
from jax.experimental.pallas import tpu as pltpu
from jax.experimental.pallas import tpu_sc as plsc

F32 = jnp.float32
I32 = jnp.int32

N = 10000
E = 320000
D_IN = 128
H = 8
F = 64
D_MID = H * F

NC = 2
NS = 16
NWORK = NC * NS

NP = 10240
RPT = NP // NS
EPT = E // NWORK
W = 80
NWIN = EPT // W

RB = 1024
NB = NP // RB

_MESH = dict(core_axis_name="c", subcore_axis_name="s")

_GDN = jax.lax.GatherDimensionNumbers(
    offset_dims=(), collapsed_slice_dims=(0,), start_index_map=(0,))


def _permute16(v, idx):
    return jax.lax.gather(
        v, idx[:, None], _GDN, slice_sizes=(1,),
        mode=jax.lax.GatherScatterMode.PROMISE_IN_BOUNDS)


HP = H // 2
FP = 2 * F


def _proj_body(x_ref, wl_ref, bl_ref, wr_ref, br_ref, xl_ref, xr_ref):
    xb = x_ref[...]
    xl = jnp.dot(xb, wl_ref[...], preferred_element_type=F32) + bl_ref[...]
    xr = jnp.dot(xb, wr_ref[...], preferred_element_type=F32) + br_ref[...]
    for h in range(H):
        xl_ref[h] = xl[:, h * F:(h + 1) * F]
        xr_ref[h] = xr[:, h * F:(h + 1) * F]


def _proj(xp, Wl1, bl1r, Wr1, br1r):
    return pl.pallas_call(
        _proj_body,
        grid=(NB,),
        in_specs=[
            pl.BlockSpec((RB, D_IN), lambda i: (i, 0)),
            pl.BlockSpec((D_IN, D_MID), lambda i: (0, 0)),
            pl.BlockSpec((1, D_MID), lambda i: (0, 0)),
            pl.BlockSpec((D_IN, D_MID), lambda i: (0, 0)),
            pl.BlockSpec((1, D_MID), lambda i: (0, 0)),
        ],
        out_specs=[
            pl.BlockSpec((H, RB, F), lambda i: (0, i, 0)),
            pl.BlockSpec((H, RB, F), lambda i: (0, i, 0)),
        ],
        out_shape=[
            jax.ShapeDtypeStruct((H, NP, F), F32),
            jax.ShapeDtypeStruct((H, NP, F), F32),
        ],
    )(xp, Wl1, bl1r, Wr1, br1r)


NBUF = 3
EPT2 = E // NS
NWIN2 = EPT2 // W


def _edge1_body(xl_hbm, xr_hbm, att_hbm, src_hbm, dst_hbm,
                acc_out, den_out,
                shacc, shden, att_v,
                sidx_v, didx_v, didxg_v, a_v, b_v, ex_v, t_v, isem, gsem):
    c = lax.axis_index("c")
    s = lax.axis_index("s")
    base = s * EPT2
    row0 = s * RPT
    cnp = c * NP

    z16 = jnp.zeros((16,), F32)

    def _zrow(r, _):
        for k in range(F // 16):
            a_v[0, r, pl.ds(k * 16, 16)] = z16
        return _

    lax.fori_loop(0, W, _zrow, None)

    def _zden(r, _):
        ex_v[0, pl.ds(r * 16, 16)] = z16
        return _

    lax.fori_loop(0, W // 16, _zden, None)

    for q in range(RPT // W):
        pltpu.sync_copy(a_v.at[0], shacc.at[pl.ds(row0 + q * W, W)])
        pltpu.sync_copy(ex_v.at[0], shden.at[pl.ds(row0 + q * W, W)])
    pltpu.sync_copy(att_hbm, att_v)
    plsc.subcore_barrier()

    nk = F // 16
    attv = [att_v[pl.ds(c * F + k * 16, 16)] for k in range(nk)]
    iota16 = jax.lax.iota(I32, 16)

    def _lidx(w, q):
        e0 = base + w * W
        pltpu.async_copy(src_hbm.at[pl.ds(e0, W)], sidx_v.at[q], isem.at[q])
        pltpu.async_copy(dst_hbm.at[pl.ds(e0, W)], didx_v.at[q], isem.at[q])

    def _didx(q):
        pltpu.make_async_copy(
            src_hbm.at[pl.ds(0, W)], sidx_v.at[q], isem.at[q]).wait()
        pltpu.make_async_copy(
            src_hbm.at[pl.ds(0, W)], didx_v.at[q], isem.at[q]).wait()

    def _shift(q):
        for t in range(W // 16):
            sl = pl.ds(t * 16, 16)
            sidx_v[q, sl] = sidx_v[q, sl] + cnp
            didxg_v[q, sl] = didx_v[q, sl] + cnp

    def _gissue(q):
        pltpu.async_copy(xl_hbm.at[sidx_v.at[q]], a_v.at[q], gsem.at[q])
        pltpu.async_copy(xr_hbm.at[didxg_v.at[q]], b_v.at[q], gsem.at[q])

    def _gdrain(q):
        pltpu.make_async_copy(
            xl_hbm.at[sidx_v.at[q]], a_v.at[q], gsem.at[q]).wait()
        pltpu.make_async_copy(
            xr_hbm.at[didxg_v.at[q]], b_v.at[q], gsem.at[q]).wait()

    def _compute(q):
        def _group(g, _):
            r0 = g * 16
            for e in range(16):
                sv = None
                for k in range(nk):
                    av = a_v[q, r0 + e, pl.ds(k * 16, 16)]
                    bv = b_v[q, r0 + e, pl.ds(k * 16, 16)]
                    z = av + bv
                    m = jnp.maximum(z, 0.2 * z)
                    tk = m * attv[k]
                    sv = tk if sv is None else sv + tk
                t_v[e, pl.ds(0, 16)] = sv
            alpha = None
            for l in range(16):
                col = plsc.load_gather(t_v, [iota16, jnp.full((16,), l, I32)])
                alpha = col if alpha is None else alpha + col
            ex = jnp.exp(alpha)
            ex_v[q, pl.ds(r0, 16)] = ex
            for e in range(16):
                es = ex[e]
                for k in range(nk):
                    a_v[q, r0 + e, pl.ds(k * 16, 16)] = (
                        a_v[q, r0 + e, pl.ds(k * 16, 16)] * es)
            return _

        lax.fori_loop(0, W // 16, _group, None)

    def _scatter(q):
        pltpu.sync_copy(ex_v.at[q], shden.at[didx_v.at[q]], add=True)
        pltpu.sync_copy(a_v.at[q], shacc.at[didx_v.at[q]], add=True)

    _lidx(0, 0)
    _lidx(1, 1)
    _didx(0)
    _shift(0)
    _gissue(0)
    _lidx(2, 2)

    NJ = (NWIN2 - 4) // NBUF
    assert NWIN2 - NBUF * NJ == 4

    def _iter3(j, carry):
        w0 = NBUF * j
        for i in range(NBUF):
            q = i
            q1 = (i + 1) % NBUF
            _didx(q1)
            _shift(q1)
            _gissue(q1)
            _gdrain(q)
            _compute(q)
            _scatter(q)
            _lidx(w0 + i + NBUF, q)
        return carry

    lax.fori_loop(0, NJ, _iter3, 0)

    for w in range(NBUF * NJ, NWIN2):
        q = w % NBUF
        if w + 1 < NWIN2:
            q1 = (w + 1) % NBUF
            _didx(q1)
            _shift(q1)
            _gissue(q1)
        _gdrain(q)
        _compute(q)
        _scatter(q)
        if w + NBUF < NWIN2:
            _lidx(w + NBUF, q)

    plsc.subcore_barrier()
    out0 = cnp + row0
    pltpu.sync_copy(shacc.at[pl.ds(row0, RPT)], acc_out.at[pl.ds(out0, RPT)])
    pltpu.sync_copy(shden.at[pl.ds(row0, RPT)], den_out.at[pl.ds(out0, RPT)])


def _edge1(xl_p, xr_p, att_p, src, dst):
    k = functools.partial(
        pl.kernel,
        out_type=(
            jax.ShapeDtypeStruct((NC * NP, F), F32),
            jax.ShapeDtypeStruct((NC * NP,), F32),
        ),
        mesh=plsc.VectorSubcoreMesh(**_MESH),
        compiler_params=pltpu.CompilerParams(needs_layout_passes=False, use_tc_tiling_on_sc=False),
        scratch_types=[
            pltpu.VMEM_SHARED((NP, F), F32),
            pltpu.VMEM_SHARED((NP,), F32),
            pltpu.VMEM((2 * F,), F32),
            pltpu.VMEM((NBUF, W), I32),
            pltpu.VMEM((NBUF, W), I32),
            pltpu.VMEM((NBUF, W), I32),
            pltpu.VMEM((NBUF, W, F), F32),
            pltpu.VMEM((NBUF, W, F), F32),
            pltpu.VMEM((NBUF, W), F32),
            pltpu.VMEM((16, 16), F32),
            pltpu.SemaphoreType.DMA((NBUF,)),
            pltpu.SemaphoreType.DMA((NBUF,)),
        ],
    )(_edge1_body)
    return k(xl_p, xr_p, att_p, src, dst)


def _combine_body(acc_ref, den_ref, bias_ref, wl2_ref, wr2_ref, sc_ref,
                  lr2_ref):
    rden = 1.0 / (den_ref[...] + 1e-16)
    xl2 = None
    xr2 = None
    for h in range(H):
        piece = acc_ref[h] * rden[h][:, None] + bias_ref[h][None, :]
        piece = jnp.maximum(piece, 0.0)
        cl = jnp.sum(piece * wl2_ref[h][None, :], axis=1)
        cr = jnp.sum(piece * wr2_ref[h][None, :], axis=1)
        xl2 = cl if xl2 is None else xl2 + cl
        xr2 = cr if xr2 is None else xr2 + cr
    lr2_ref[0, :] = xl2 + sc_ref[0, 0]
    lr2_ref[1, :] = xr2 + sc_ref[0, 1]


def _combine(accs, dens, bias1hf, wl2hf, wr2hf, sc2):
    return pl.pallas_call(
        _combine_body,
        grid=(NB,),
        in_specs=[
            pl.BlockSpec((H, RB, F), lambda i: (0, i, 0)),
            pl.BlockSpec((H, RB), lambda i: (0, i)),
            pl.BlockSpec((H, F), lambda i: (0, 0)),
            pl.BlockSpec((H, F), lambda i: (0, 0)),
            pl.BlockSpec((H, F), lambda i: (0, 0)),
            pl.BlockSpec((1, 8), lambda i: (0, 0)),
        ],
        out_specs=pl.BlockSpec((NC, RB), lambda i: (0, i)),
        out_shape=jax.ShapeDtypeStruct((NC, NP), F32),
    )(accs, dens, bias1hf, wl2hf, wr2hf, sc2)


def _edge2_body(xl2_hbm, xr2_hbm, att_hbm, src_hbm, dst_hbm,
                num_out, den_out, ex_out,
                shnum, shden, xl2_v, xr2_v, att_v, sidx_v, didx_v,
                ex_v, nm_v, zden_v):
    c = lax.axis_index("c")
    s = lax.axis_index("s")
    wid = c * NS + s
    base = wid * EPT
    row0 = s * RPT

    z16 = jnp.zeros((16,), F32)

    def _zden(r, _):
        zden_v[pl.ds(r * 16, 16)] = z16
        return _

    lax.fori_loop(0, RPT // 16, _zden, None)
    pltpu.sync_copy(zden_v, shnum.at[pl.ds(row0, RPT)])
    pltpu.sync_copy(zden_v, shden.at[pl.ds(row0, RPT)])
    pltpu.sync_copy(xl2_hbm, xl2_v)
    pltpu.sync_copy(xr2_hbm, xr2_v)
    pltpu.sync_copy(att_hbm, att_v)
    plsc.subcore_barrier()

    atts = att_v[pl.ds(0, 16)][0]

    def _window(w, _):
        e0 = base + w * W
        pltpu.sync_copy(src_hbm.at[pl.ds(e0, W)], sidx_v)
        pltpu.sync_copy(dst_hbm.at[pl.ds(e0, W)], didx_v)

        def _group(g, _):
            r0 = g * 16
            sv = sidx_v[pl.ds(r0, 16)]
            dv = didx_v[pl.ds(r0, 16)]
            xls = plsc.load_gather(xl2_v, [sv])
            xrd = plsc.load_gather(xr2_v, [dv])
            z = xls + xrd
            m = jnp.maximum(z, 0.2 * z)
            ex = jnp.exp(atts * m)
            ex_v[pl.ds(r0, 16)] = ex
            nm_v[pl.ds(r0, 16)] = ex * xls
            return _

        lax.fori_loop(0, W // 16, _group, None)

        pltpu.sync_copy(ex_v, ex_out.at[pl.ds(e0, W)])
        pltpu.sync_copy(ex_v, shden.at[didx_v], add=True)
        pltpu.sync_copy(nm_v, shnum.at[didx_v], add=True)
        return _

    lax.fori_loop(0, NWIN, _window, None)
    plsc.subcore_barrier()

    out0 = c * NP + row0
    pltpu.sync_copy(shnum.at[pl.ds(row0, RPT)], num_out.at[pl.ds(out0, RPT)])
    pltpu.sync_copy(shden.at[pl.ds(row0, RPT)], den_out.at[pl.ds(out0, RPT)])


def _edge2(xl2, xr2, att2p, src, dst):
    k = functools.partial(
        pl.kernel,
        out_type=(
            jax.ShapeDtypeStruct((NC * NP,), F32),
            jax.ShapeDtypeStruct((NC * NP,), F32),
            jax.ShapeDtypeStruct((E,), F32),
        ),
        mesh=plsc.VectorSubcoreMesh(**_MESH),
        compiler_params=pltpu.CompilerParams(needs_layout_passes=False, use_tc_tiling_on_sc=False),
        scratch_types=[
            pltpu.VMEM_SHARED((NP,), F32),
            pltpu.VMEM_SHARED((NP,), F32),
            pltpu.VMEM((NP,), F32),
            pltpu.VMEM((NP,), F32),
            pltpu.VMEM((16,), F32),
            pltpu.VMEM((W,), I32),
            pltpu.VMEM((W,), I32),
            pltpu.VMEM((W,), F32),
            pltpu.VMEM((W,), F32),
            pltpu.VMEM((RPT,), F32),
        ],
    )(_edge2_body)
    return k(xl2, xr2, att2p, src, dst)


def _final2_body(num_ref, den_ref, sc_ref, h2_ref, rden_ref):
    dent = den_ref[0] + den_ref[1]
    numt = num_ref[0] + num_ref[1]
    r = 1.0 / (dent + 1e-16)
    h2_ref[0, :] = numt * r + sc_ref[0, 2]
    rden_ref[0, :] = r


def _final2(num2, den2, sc2):
    return pl.pallas_call(
        _final2_body,
        grid=(NB,),
        in_specs=[
            pl.BlockSpec((NC, RB), lambda i: (0, i)),
            pl.BlockSpec((NC, RB), lambda i: (0, i)),
            pl.BlockSpec((1, 8), lambda i: (0, 0)),
        ],
        out_specs=[
            pl.BlockSpec((1, RB), lambda i: (0, i)),
            pl.BlockSpec((1, RB), lambda i: (0, i)),
        ],
        out_shape=[
            jax.ShapeDtypeStruct((1, NP), F32),
            jax.ShapeDtypeStruct((1, NP), F32),
        ],
    )(num2, den2, sc2)


def _edge2b_body(rden_hbm, ex_hbm, dst_hbm, a2_out,
                 rden_v, didx_v, ex_v, a_v):
    c = lax.axis_index("c")
    s = lax.axis_index("s")
    base = (c * NS + s) * EPT
    pltpu.sync_copy(rden_hbm, rden_v)

    def _window(w, _):
        e0 = base + w * W
        pltpu.sync_copy(dst_hbm.at[pl.ds(e0, W)], didx_v)
        pltpu.sync_copy(ex_hbm.at[pl.ds(e0, W)], ex_v)

        def _group(g, _):
            r0 = g * 16
            dv = didx_v[pl.ds(r0, 16)]
            r = plsc.load_gather(rden_v, [dv])
            a_v[pl.ds(r0, 16)] = ex_v[pl.ds(r0, 16)] * r
            return _

        lax.fori_loop(0, W // 16, _group, None)
        pltpu.sync_copy(a_v, a2_out.at[pl.ds(e0, W)])
        return _

    lax.fori_loop(0, NWIN, _window, None)


def _edge2b(rden, ex2, dst):
    k = functools.partial(
        pl.kernel,
        out_type=jax.ShapeDtypeStruct((E,), F32),
        mesh=plsc.VectorSubcoreMesh(**_MESH),
        compiler_params=pltpu.CompilerParams(needs_layout_passes=False, use_tc_tiling_on_sc=False),
        scratch_types=[
            pltpu.VMEM((NP,), F32),
            pltpu.VMEM((W,), I32),
            pltpu.VMEM((W,), F32),
            pltpu.VMEM((W,), F32),
        ],
    )(_edge2b_body)
    return k(rden, ex2, dst)


def kernel(x, edge_index, Wl1, bl1, Wr1, br1, att1, bias1,
           Wl2, bl2, Wr2, br2, att2, bias2):
    xp = jnp.pad(x, ((0, NP - N), (0, 0)))
    src = edge_index[0]
    dst = edge_index[1]

    XL, XR = _proj(xp, Wl1, bl1.reshape(1, D_MID), Wr1, br1.reshape(1, D_MID))

    XLp = XL.reshape(HP, NC * NP, F)
    XRp = XR.reshape(HP, NC * NP, F)
    att1p = att1.reshape(HP, 2 * F)
    accs = []
    dens = []
    for p in range(HP):
        acc_p, den_p = _edge1(XLp[p], XRp[p], att1p[p], src, dst)
        accs.append(acc_p.reshape(NC, NP, F))
        dens.append(den_p.reshape(NC, NP))
    accs = jnp.concatenate(accs)
    dens = jnp.concatenate(dens)

    sc2 = jnp.stack([bl2[0], br2[0], bias2[0], att2[0, 0],
                     0.0, 0.0, 0.0, 0.0]).reshape(1, 8).astype(F32)
    att2p = jnp.pad(att2.reshape(-1), (0, 15)).astype(F32)

    lr2 = _combine(accs, dens, bias1.reshape(H, F),
                   Wl2.reshape(H, F), Wr2.reshape(H, F), sc2)

    num2f, den2f, ex2 = _edge2(lr2[0], lr2[1], att2p, src, dst)
    num2 = num2f.reshape(NC, NP)
    den2 = den2f.reshape(NC, NP)

    h2row, rdenrow = _final2(num2, den2, sc2)
    a2 = _edge2b(rdenrow.reshape(NP), ex2, dst)

    h2 = h2row[0, :N].reshape(N, 1)
    return (h2, edge_index, a2.reshape(E, 1))


# speedup vs baseline: 1.7348x; 1.1488x over previous
import functools

import jax
import jax.numpy as jnp
from jax import lax
from jax.experimental import pallas as pl
from jax.experimental.pallas import tpu as pltpu
from jax.experimental.pallas import tpu_sc as plsc

F32 = jnp.float32
I32 = jnp.int32

N = 10000
E = 320000
D_IN = 128
H = 8
F = 64
D_MID = H * F

NC = 2
NS = 16
NWORK = NC * NS

NP = 10240
RPT = NP // NS
EPT = E // NWORK
W = 80
NWIN = EPT // W

RB = 1024
NB = NP // RB

_MESH = dict(core_axis_name="c", subcore_axis_name="s")

_GDN = jax.lax.GatherDimensionNumbers(
    offset_dims=(), collapsed_slice_dims=(0,), start_index_map=(0,))


def _permute16(v, idx):
    return jax.lax.gather(
        v, idx[:, None], _GDN, slice_sizes=(1,),
        mode=jax.lax.GatherScatterMode.PROMISE_IN_BOUNDS)


HP = H // 2
FP = 2 * F


def _proj_body(x_ref, wl_ref, bl_ref, wr_ref, br_ref, xl_ref, xr_ref):
    xb = x_ref[...]
    xl = jnp.dot(xb, wl_ref[...], preferred_element_type=F32) + bl_ref[...]
    xr = jnp.dot(xb, wr_ref[...], preferred_element_type=F32) + br_ref[...]
    for h in range(H):
        xl_ref[h] = xl[:, h * F:(h + 1) * F]
        xr_ref[h] = xr[:, h * F:(h + 1) * F]


def _proj(xp, Wl1, bl1r, Wr1, br1r):
    return pl.pallas_call(
        _proj_body,
        grid=(NB,),
        in_specs=[
            pl.BlockSpec((RB, D_IN), lambda i: (i, 0)),
            pl.BlockSpec((D_IN, D_MID), lambda i: (0, 0)),
            pl.BlockSpec((1, D_MID), lambda i: (0, 0)),
            pl.BlockSpec((D_IN, D_MID), lambda i: (0, 0)),
            pl.BlockSpec((1, D_MID), lambda i: (0, 0)),
        ],
        out_specs=[
            pl.BlockSpec((H, RB, F), lambda i: (0, i, 0)),
            pl.BlockSpec((H, RB, F), lambda i: (0, i, 0)),
        ],
        out_shape=[
            jax.ShapeDtypeStruct((H, NP, F), F32),
            jax.ShapeDtypeStruct((H, NP, F), F32),
        ],
    )(xp, Wl1, bl1r, Wr1, br1r)


NBUF = 3
EPT2 = E // NS
NWIN2 = EPT2 // W


def _edge1_body(xl_hbm, xr_hbm, att_hbm, src_hbm, dst_hbm,
                acc_out, den_out,
                shacc, shden, att_v,
                sidx_v, didx_v, didxg_v, didxs_v, a_v, b_v, ex_v, t_v,
                isem, gsem, ssem):
    c = lax.axis_index("c")
    s = lax.axis_index("s")
    base = s * EPT2
    row0 = s * RPT
    cnp = c * NP

    z16 = jnp.zeros((16,), F32)

    def _zrow(r, _):
        for k in range(F // 16):
            a_v[0, r, pl.ds(k * 16, 16)] = z16
        return _

    lax.fori_loop(0, W, _zrow, None)

    def _zden(r, _):
        ex_v[0, pl.ds(r * 16, 16)] = z16
        return _

    lax.fori_loop(0, W // 16, _zden, None)

    for q in range(RPT // W):
        pltpu.sync_copy(a_v.at[0], shacc.at[pl.ds(row0 + q * W, W)])
        pltpu.sync_copy(ex_v.at[0], shden.at[pl.ds(row0 + q * W, W)])
    pltpu.sync_copy(att_hbm, att_v)
    plsc.subcore_barrier()

    nk = F // 16
    attv = [att_v[pl.ds(c * F + k * 16, 16)] for k in range(nk)]
    iota16 = jax.lax.iota(I32, 16)

    def _lidx(w, q):
        e0 = base + w * W
        pltpu.async_copy(src_hbm.at[pl.ds(e0, W)], sidx_v.at[q], isem.at[q])
        pltpu.async_copy(dst_hbm.at[pl.ds(e0, W)], didx_v.at[q], isem.at[q])

    def _didx(q):
        pltpu.make_async_copy(
            src_hbm.at[pl.ds(0, W)], sidx_v.at[q], isem.at[q]).wait()
        pltpu.make_async_copy(
            src_hbm.at[pl.ds(0, W)], didx_v.at[q], isem.at[q]).wait()

    def _shift(q):
        for t in range(W // 16):
            sl = pl.ds(t * 16, 16)
            dv = didx_v[q, sl]
            sidx_v[q, sl] = sidx_v[q, sl] + cnp
            didxg_v[q, sl] = dv + cnp
            didxs_v[q, sl] = dv

    def _gissue(q):
        pltpu.async_copy(xl_hbm.at[sidx_v.at[q]], a_v.at[q], gsem.at[q])
        pltpu.async_copy(xr_hbm.at[didxg_v.at[q]], b_v.at[q], gsem.at[q])

    def _gdrain(q):
        pltpu.make_async_copy(
            xl_hbm.at[sidx_v.at[q]], a_v.at[q], gsem.at[q]).wait()
        pltpu.make_async_copy(
            xr_hbm.at[didxg_v.at[q]], b_v.at[q], gsem.at[q]).wait()

    def _compute(q):
        def _group(g, _):
            r0 = g * 16
            for e in range(16):
                sv = None
                for k in range(nk):
                    av = a_v[q, r0 + e, pl.ds(k * 16, 16)]
                    bv = b_v[q, r0 + e, pl.ds(k * 16, 16)]
                    z = av + bv
                    m = jnp.maximum(z, 0.2 * z)
                    tk = m * attv[k]
                    sv = tk if sv is None else sv + tk
                t_v[e, pl.ds(0, 16)] = sv
            alpha = None
            for l in range(16):
                col = plsc.load_gather(t_v, [iota16, jnp.full((16,), l, I32)])
                alpha = col if alpha is None else alpha + col
            ex = jnp.exp(alpha)
            ex_v[q, pl.ds(r0, 16)] = ex
            for e in range(16):
                es = ex[e]
                for k in range(nk):
                    a_v[q, r0 + e, pl.ds(k * 16, 16)] = (
                        a_v[q, r0 + e, pl.ds(k * 16, 16)] * es)
            return _

        lax.fori_loop(0, W // 16, _group, None)

    def _scatter(q):
        pltpu.async_copy(ex_v.at[q], shden.at[didxs_v.at[q]], ssem.at[q],
                         add=True)
        pltpu.async_copy(a_v.at[q], shacc.at[didxs_v.at[q]], ssem.at[q],
                         add=True)

    def _sdrain(q):
        pltpu.make_async_copy(
            ex_v.at[q], shden.at[didxs_v.at[q]], ssem.at[q]).wait()
        pltpu.make_async_copy(
            a_v.at[q], shacc.at[didxs_v.at[q]], ssem.at[q]).wait()

    def _blockA(q1, do_sdrain):
        _didx(q1)
        if do_sdrain:
            _sdrain(q1)
        _shift(q1)
        _gissue(q1)

    def _blockB(q):
        _gdrain(q)
        _compute(q)
        _scatter(q)

    _lidx(0, 0)
    _lidx(1, 1)
    _didx(0)
    _shift(0)
    _gissue(0)
    _lidx(2, 2)

    NJ = (NWIN2 - 4) // NBUF
    assert NWIN2 - NBUF * NJ == 4

    for i in range(NBUF):
        _blockA((i + 1) % NBUF, do_sdrain=(i == NBUF - 1))
        _blockB(i)
        _lidx(i + NBUF, i)

    def _iter3(j, carry):
        w0 = NBUF * j
        for i in range(NBUF):
            _blockA((i + 1) % NBUF, do_sdrain=True)
            _blockB(i)
            _lidx(w0 + i + NBUF, i)
        return carry

    lax.fori_loop(1, NJ, _iter3, 0)

    for w in range(NBUF * NJ, NWIN2):
        q = w % NBUF
        if w + 1 < NWIN2:
            _blockA((w + 1) % NBUF, do_sdrain=True)
        _blockB(q)
        if w + NBUF < NWIN2:
            _lidx(w + NBUF, q)

    for w in range(NWIN2 - NBUF, NWIN2):
        _sdrain(w % NBUF)

    plsc.subcore_barrier()
    out0 = cnp + row0
    pltpu.sync_copy(shacc.at[pl.ds(row0, RPT)], acc_out.at[pl.ds(out0, RPT)])
    pltpu.sync_copy(shden.at[pl.ds(row0, RPT)], den_out.at[pl.ds(out0, RPT)])


def _edge1(xl_p, xr_p, att_p, src, dst):
    k = functools.partial(
        pl.kernel,
        out_type=(
            jax.ShapeDtypeStruct((NC * NP, F), F32),
            jax.ShapeDtypeStruct((NC * NP,), F32),
        ),
        mesh=plsc.VectorSubcoreMesh(**_MESH),
        compiler_params=pltpu.CompilerParams(needs_layout_passes=False, use_tc_tiling_on_sc=False),
        scratch_types=[
            pltpu.VMEM_SHARED((NP, F), F32),
            pltpu.VMEM_SHARED((NP,), F32),
            pltpu.VMEM((2 * F,), F32),
            pltpu.VMEM((NBUF, W), I32),
            pltpu.VMEM((NBUF, W), I32),
            pltpu.VMEM((NBUF, W), I32),
            pltpu.VMEM((NBUF, W), I32),
            pltpu.VMEM((NBUF, W, F), F32),
            pltpu.VMEM((NBUF, W, F), F32),
            pltpu.VMEM((NBUF, W), F32),
            pltpu.VMEM((16, 16), F32),
            pltpu.SemaphoreType.DMA((NBUF,)),
            pltpu.SemaphoreType.DMA((NBUF,)),
            pltpu.SemaphoreType.DMA((NBUF,)),
        ],
    )(_edge1_body)
    return k(xl_p, xr_p, att_p, src, dst)


def _combine_body(acc_ref, den_ref, bias_ref, wl2_ref, wr2_ref, sc_ref,
                  lr2_ref):
    rden = 1.0 / (den_ref[...] + 1e-16)
    xl2 = None
    xr2 = None
    for h in range(H):
        piece = acc_ref[h] * rden[h][:, None] + bias_ref[h][None, :]
        piece = jnp.maximum(piece, 0.0)
        cl = jnp.sum(piece * wl2_ref[h][None, :], axis=1)
        cr = jnp.sum(piece * wr2_ref[h][None, :], axis=1)
        xl2 = cl if xl2 is None else xl2 + cl
        xr2 = cr if xr2 is None else xr2 + cr
    lr2_ref[0, :] = xl2 + sc_ref[0, 0]
    lr2_ref[1, :] = xr2 + sc_ref[0, 1]


def _combine(accs, dens, bias1hf, wl2hf, wr2hf, sc2):
    return pl.pallas_call(
        _combine_body,
        grid=(NB,),
        in_specs=[
            pl.BlockSpec((H, RB, F), lambda i: (0, i, 0)),
            pl.BlockSpec((H, RB), lambda i: (0, i)),
            pl.BlockSpec((H, F), lambda i: (0, 0)),
            pl.BlockSpec((H, F), lambda i: (0, 0)),
            pl.BlockSpec((H, F), lambda i: (0, 0)),
            pl.BlockSpec((1, 8), lambda i: (0, 0)),
        ],
        out_specs=pl.BlockSpec((NC, RB), lambda i: (0, i)),
        out_shape=jax.ShapeDtypeStruct((NC, NP), F32),
    )(accs, dens, bias1hf, wl2hf, wr2hf, sc2)


def _edge2_body(xl2_hbm, xr2_hbm, att_hbm, src_hbm, dst_hbm,
                num_out, den_out, ex_out,
                shnum, shden, xl2_v, xr2_v, att_v, sidx_v, didx_v,
                ex_v, nm_v, zden_v):
    c = lax.axis_index("c")
    s = lax.axis_index("s")
    wid = c * NS + s
    base = wid * EPT
    row0 = s * RPT

    z16 = jnp.zeros((16,), F32)

    def _zden(r, _):
        zden_v[pl.ds(r * 16, 16)] = z16
        return _

    lax.fori_loop(0, RPT // 16, _zden, None)
    pltpu.sync_copy(zden_v, shnum.at[pl.ds(row0, RPT)])
    pltpu.sync_copy(zden_v, shden.at[pl.ds(row0, RPT)])
    pltpu.sync_copy(xl2_hbm, xl2_v)
    pltpu.sync_copy(xr2_hbm, xr2_v)
    pltpu.sync_copy(att_hbm, att_v)
    plsc.subcore_barrier()

    atts = att_v[pl.ds(0, 16)][0]

    def _window(w, _):
        e0 = base + w * W
        pltpu.sync_copy(src_hbm.at[pl.ds(e0, W)], sidx_v)
        pltpu.sync_copy(dst_hbm.at[pl.ds(e0, W)], didx_v)

        def _group(g, _):
            r0 = g * 16
            sv = sidx_v[pl.ds(r0, 16)]
            dv = didx_v[pl.ds(r0, 16)]
            xls = plsc.load_gather(xl2_v, [sv])
            xrd = plsc.load_gather(xr2_v, [dv])
            z = xls + xrd
            m = jnp.maximum(z, 0.2 * z)
            ex = jnp.exp(atts * m)
            ex_v[pl.ds(r0, 16)] = ex
            nm_v[pl.ds(r0, 16)] = ex * xls
            return _

        lax.fori_loop(0, W // 16, _group, None)

        pltpu.sync_copy(ex_v, ex_out.at[pl.ds(e0, W)])
        pltpu.sync_copy(ex_v, shden.at[didx_v], add=True)
        pltpu.sync_copy(nm_v, shnum.at[didx_v], add=True)
        return _

    lax.fori_loop(0, NWIN, _window, None)
    plsc.subcore_barrier()

    out0 = c * NP + row0
    pltpu.sync_copy(shnum.at[pl.ds(row0, RPT)], num_out.at[pl.ds(out0, RPT)])
    pltpu.sync_copy(shden.at[pl.ds(row0, RPT)], den_out.at[pl.ds(out0, RPT)])


def _edge2(xl2, xr2, att2p, src, dst):
    k = functools.partial(
        pl.kernel,
        out_type=(
            jax.ShapeDtypeStruct((NC * NP,), F32),
            jax.ShapeDtypeStruct((NC * NP,), F32),
            jax.ShapeDtypeStruct((E,), F32),
        ),
        mesh=plsc.VectorSubcoreMesh(**_MESH),
        compiler_params=pltpu.CompilerParams(needs_layout_passes=False, use_tc_tiling_on_sc=False),
        scratch_types=[
            pltpu.VMEM_SHARED((NP,), F32),
            pltpu.VMEM_SHARED((NP,), F32),
            pltpu.VMEM((NP,), F32),
            pltpu.VMEM((NP,), F32),
            pltpu.VMEM((16,), F32),
            pltpu.VMEM((W,), I32),
            pltpu.VMEM((W,), I32),
            pltpu.VMEM((W,), F32),
            pltpu.VMEM((W,), F32),
            pltpu.VMEM((RPT,), F32),
        ],
    )(_edge2_body)
    return k(xl2, xr2, att2p, src, dst)


def _final2_body(num_ref, den_ref, sc_ref, h2_ref, rden_ref):
    dent = den_ref[0] + den_ref[1]
    numt = num_ref[0] + num_ref[1]
    r = 1.0 / (dent + 1e-16)
    h2_ref[0, :] = numt * r + sc_ref[0, 2]
    rden_ref[0, :] = r


def _final2(num2, den2, sc2):
    return pl.pallas_call(
        _final2_body,
        grid=(NB,),
        in_specs=[
            pl.BlockSpec((NC, RB), lambda i: (0, i)),
            pl.BlockSpec((NC, RB), lambda i: (0, i)),
            pl.BlockSpec((1, 8), lambda i: (0, 0)),
        ],
        out_specs=[
            pl.BlockSpec((1, RB), lambda i: (0, i)),
            pl.BlockSpec((1, RB), lambda i: (0, i)),
        ],
        out_shape=[
            jax.ShapeDtypeStruct((1, NP), F32),
            jax.ShapeDtypeStruct((1, NP), F32),
        ],
    )(num2, den2, sc2)


def _edge2b_body(rden_hbm, ex_hbm, dst_hbm, a2_out,
                 rden_v, didx_v, ex_v, a_v):
    c = lax.axis_index("c")
    s = lax.axis_index("s")
    base = (c * NS + s) * EPT
    pltpu.sync_copy(rden_hbm, rden_v)

    def _window(w, _):
        e0 = base + w * W
        pltpu.sync_copy(dst_hbm.at[pl.ds(e0, W)], didx_v)
        pltpu.sync_copy(ex_hbm.at[pl.ds(e0, W)], ex_v)

        def _group(g, _):
            r0 = g * 16
            dv = didx_v[pl.ds(r0, 16)]
            r = plsc.load_gather(rden_v, [dv])
            a_v[pl.ds(r0, 16)] = ex_v[pl.ds(r0, 16)] * r
            return _

        lax.fori_loop(0, W // 16, _group, None)
        pltpu.sync_copy(a_v, a2_out.at[pl.ds(e0, W)])
        return _

    lax.fori_loop(0, NWIN, _window, None)


def _edge2b(rden, ex2, dst):
    k = functools.partial(
        pl.kernel,
        out_type=jax.ShapeDtypeStruct((E,), F32),
        mesh=plsc.VectorSubcoreMesh(**_MESH),
        compiler_params=pltpu.CompilerParams(needs_layout_passes=False, use_tc_tiling_on_sc=False),
        scratch_types=[
            pltpu.VMEM((NP,), F32),
            pltpu.VMEM((W,), I32),
            pltpu.VMEM((W,), F32),
            pltpu.VMEM((W,), F32),
        ],
    )(_edge2b_body)
    return k(rden, ex2, dst)


def kernel(x, edge_index, Wl1, bl1, Wr1, br1, att1, bias1,
           Wl2, bl2, Wr2, br2, att2, bias2):
    xp = jnp.pad(x, ((0, NP - N), (0, 0)))
    src = edge_index[0]
    dst = edge_index[1]

    XL, XR = _proj(xp, Wl1, bl1.reshape(1, D_MID), Wr1, br1.reshape(1, D_MID))

    XLp = XL.reshape(HP, NC * NP, F)
    XRp = XR.reshape(HP, NC * NP, F)
    att1p = att1.reshape(HP, 2 * F)
    accs = []
    dens = []
    for p in range(HP):
        acc_p, den_p = _edge1(XLp[p], XRp[p], att1p[p], src, dst)
        accs.append(acc_p.reshape(NC, NP, F))
        dens.append(den_p.reshape(NC, NP))
    accs = jnp.concatenate(accs)
    dens = jnp.concatenate(dens)

    sc2 = jnp.stack([bl2[0], br2[0], bias2[0], att2[0, 0],
                     0.0, 0.0, 0.0, 0.0]).reshape(1, 8).astype(F32)
    att2p = jnp.pad(att2.reshape(-1), (0, 15)).astype(F32)

    lr2 = _combine(accs, dens, bias1.reshape(H, F),
                   Wl2.reshape(H, F), Wr2.reshape(H, F), sc2)

    num2f, den2f, ex2 = _edge2(lr2[0], lr2[1], att2p, src, dst)
    num2 = num2f.reshape(NC, NP)
    den2 = den2f.reshape(NC, NP)

    h2row, rdenrow = _final2(num2, den2, sc2)
    a2 = _edge2b(rdenrow.reshape(NP), ex2, dst)

    h2 = h2row[0, :N].reshape(N, 1)
    return (h2, edge_index, a2.reshape(E, 1))


# speedup vs baseline: 3.1064x; 1.7907x over previous
import functools

import jax
import jax.numpy as jnp
from jax import lax
from jax.experimental import pallas as pl
from jax.experimental.pallas import tpu as pltpu
from jax.experimental.pallas import tpu_sc as plsc

F32 = jnp.float32
I32 = jnp.int32

N = 10000
E = 320000
D_IN = 128
H = 8
F = 64
D_MID = H * F

NC = 2
NS = 16
NWORK = NC * NS

NP = 10240
RPT = NP // NS
EPT = E // NWORK
W = 80
NWIN = EPT // W

RB = 1024
NB = NP // RB

_MESH = dict(core_axis_name="c", subcore_axis_name="s")

_GDN = jax.lax.GatherDimensionNumbers(
    offset_dims=(), collapsed_slice_dims=(0,), start_index_map=(0,))


def _permute16(v, idx):
    return jax.lax.gather(
        v, idx[:, None], _GDN, slice_sizes=(1,),
        mode=jax.lax.GatherScatterMode.PROMISE_IN_BOUNDS)


HP = H // 2
FP = 2 * F


def _proj_body(x_ref, wl_ref, bl_ref, wr_ref, br_ref, xl_ref, xr_ref):
    xb = x_ref[...]
    xl = jnp.dot(xb, wl_ref[...], preferred_element_type=F32) + bl_ref[...]
    xr = jnp.dot(xb, wr_ref[...], preferred_element_type=F32) + br_ref[...]
    for h in range(H):
        xl_ref[h] = xl[:, h * F:(h + 1) * F]
        xr_ref[h] = xr[:, h * F:(h + 1) * F]


def _proj(xp, Wl1, bl1r, Wr1, br1r):
    return pl.pallas_call(
        _proj_body,
        grid=(NB,),
        in_specs=[
            pl.BlockSpec((RB, D_IN), lambda i: (i, 0)),
            pl.BlockSpec((D_IN, D_MID), lambda i: (0, 0)),
            pl.BlockSpec((1, D_MID), lambda i: (0, 0)),
            pl.BlockSpec((D_IN, D_MID), lambda i: (0, 0)),
            pl.BlockSpec((1, D_MID), lambda i: (0, 0)),
        ],
        out_specs=[
            pl.BlockSpec((H, RB, F), lambda i: (0, i, 0)),
            pl.BlockSpec((H, RB, F), lambda i: (0, i, 0)),
        ],
        out_shape=[
            jax.ShapeDtypeStruct((H, NP, F), F32),
            jax.ShapeDtypeStruct((H, NP, F), F32),
        ],
    )(xp, Wl1, bl1r, Wr1, br1r)


NBUF = 3
EPT2 = E // NS
NWIN2 = EPT2 // W


def _edge1_body(xl_hbm, xr_hbm, att_hbm, src_hbm, dst_hbm,
                acc_out, den_out,
                shacc, shden, att_v,
                sidx_v, didx_v, didxg_v, didxs_v, a_v, b_v, ex_v, t_v,
                isem, gsem, ssem):
    c = lax.axis_index("c")
    s = lax.axis_index("s")
    base = s * EPT2
    row0 = s * RPT
    cnp = c * NP

    z16 = jnp.zeros((16,), F32)

    def _zrow(r, _):
        for k in range(F // 16):
            a_v[0, r, pl.ds(k * 16, 16)] = z16
        return _

    lax.fori_loop(0, W, _zrow, None)

    def _zden(r, _):
        ex_v[0, pl.ds(r * 16, 16)] = z16
        return _

    lax.fori_loop(0, W // 16, _zden, None)

    for q in range(RPT // W):
        pltpu.sync_copy(a_v.at[0], shacc.at[pl.ds(row0 + q * W, W)])
        pltpu.sync_copy(ex_v.at[0], shden.at[pl.ds(row0 + q * W, W)])
    pltpu.sync_copy(att_hbm, att_v)
    plsc.subcore_barrier()

    nk = F // 16
    attv = [att_v[pl.ds(c * F + k * 16, 16)] for k in range(nk)]
    iota16 = jax.lax.iota(I32, 16)

    def _lidx(w, q):
        e0 = base + w * W
        pltpu.async_copy(src_hbm.at[pl.ds(e0, W)], sidx_v.at[q], isem.at[q])
        pltpu.async_copy(dst_hbm.at[pl.ds(e0, W)], didx_v.at[q], isem.at[q])

    def _didx(q):
        pltpu.make_async_copy(
            src_hbm.at[pl.ds(0, W)], sidx_v.at[q], isem.at[q]).wait()
        pltpu.make_async_copy(
            src_hbm.at[pl.ds(0, W)], didx_v.at[q], isem.at[q]).wait()

    def _shift(q):
        for t in range(W // 16):
            sl = pl.ds(t * 16, 16)
            dv = didx_v[q, sl]
            sidx_v[q, sl] = sidx_v[q, sl] + cnp
            didxg_v[q, sl] = dv + cnp
            didxs_v[q, sl] = dv

    def _gissue(q):
        pltpu.async_copy(xl_hbm.at[sidx_v.at[q]], a_v.at[q], gsem.at[q])
        pltpu.async_copy(xr_hbm.at[didxg_v.at[q]], b_v.at[q], gsem.at[q])

    def _gdrain(q):
        pltpu.make_async_copy(
            xl_hbm.at[sidx_v.at[q]], a_v.at[q], gsem.at[q]).wait()
        pltpu.make_async_copy(
            xr_hbm.at[didxg_v.at[q]], b_v.at[q], gsem.at[q]).wait()

    def _compute(q):
        @functools.partial(plsc.parallel_loop, 0, W // 16)
        def _group(g):
            r0 = g * 16
            for e in range(16):
                sv = None
                for k in range(nk):
                    av = a_v[q, r0 + e, pl.ds(k * 16, 16)]
                    bv = b_v[q, r0 + e, pl.ds(k * 16, 16)]
                    z = av + bv
                    m = jnp.maximum(z, 0.2 * z)
                    tk = m * attv[k]
                    sv = tk if sv is None else sv + tk
                t_v[g, e, pl.ds(0, 16)] = sv
            alpha = None
            for l in range(16):
                col = plsc.load_gather(
                    t_v, [jnp.full((16,), g, I32), iota16,
                          jnp.full((16,), l, I32)])
                alpha = col if alpha is None else alpha + col
            ex = jnp.exp(alpha)
            ex_v[q, pl.ds(r0, 16)] = ex
            for e in range(16):
                es = ex[e]
                for k in range(nk):
                    a_v[q, r0 + e, pl.ds(k * 16, 16)] = (
                        a_v[q, r0 + e, pl.ds(k * 16, 16)] * es)

    def _scatter(q):
        pltpu.async_copy(ex_v.at[q], shden.at[didxs_v.at[q]], ssem.at[q],
                         add=True)
        pltpu.async_copy(a_v.at[q], shacc.at[didxs_v.at[q]], ssem.at[q],
                         add=True)

    def _sdrain(q):
        pltpu.make_async_copy(
            ex_v.at[q], shden.at[didxs_v.at[q]], ssem.at[q]).wait()
        pltpu.make_async_copy(
            a_v.at[q], shacc.at[didxs_v.at[q]], ssem.at[q]).wait()

    def _blockA(q1, do_sdrain):
        _didx(q1)
        if do_sdrain:
            _sdrain(q1)
        _shift(q1)
        _gissue(q1)

    def _blockB(q):
        _gdrain(q)
        _compute(q)
        _scatter(q)

    _lidx(0, 0)
    _lidx(1, 1)
    _didx(0)
    _shift(0)
    _gissue(0)
    _lidx(2, 2)

    NJ = (NWIN2 - 4) // NBUF
    assert NWIN2 - NBUF * NJ == 4

    for i in range(NBUF):
        _blockA((i + 1) % NBUF, do_sdrain=(i == NBUF - 1))
        _blockB(i)
        _lidx(i + NBUF, i)

    def _iter3(j, carry):
        w0 = NBUF * j
        for i in range(NBUF):
            _blockA((i + 1) % NBUF, do_sdrain=True)
            _blockB(i)
            _lidx(w0 + i + NBUF, i)
        return carry

    lax.fori_loop(1, NJ, _iter3, 0)

    for w in range(NBUF * NJ, NWIN2):
        q = w % NBUF
        if w + 1 < NWIN2:
            _blockA((w + 1) % NBUF, do_sdrain=True)
        _blockB(q)
        if w + NBUF < NWIN2:
            _lidx(w + NBUF, q)

    for w in range(NWIN2 - NBUF, NWIN2):
        _sdrain(w % NBUF)

    plsc.subcore_barrier()
    out0 = cnp + row0
    pltpu.sync_copy(shacc.at[pl.ds(row0, RPT)], acc_out.at[pl.ds(out0, RPT)])
    pltpu.sync_copy(shden.at[pl.ds(row0, RPT)], den_out.at[pl.ds(out0, RPT)])


def _edge1(xl_p, xr_p, att_p, src, dst):
    k = functools.partial(
        pl.kernel,
        out_type=(
            jax.ShapeDtypeStruct((NC * NP, F), F32),
            jax.ShapeDtypeStruct((NC * NP,), F32),
        ),
        mesh=plsc.VectorSubcoreMesh(**_MESH),
        compiler_params=pltpu.CompilerParams(needs_layout_passes=False, use_tc_tiling_on_sc=False),
        scratch_types=[
            pltpu.VMEM_SHARED((NP, F), F32),
            pltpu.VMEM_SHARED((NP,), F32),
            pltpu.VMEM((2 * F,), F32),
            pltpu.VMEM((NBUF, W), I32),
            pltpu.VMEM((NBUF, W), I32),
            pltpu.VMEM((NBUF, W), I32),
            pltpu.VMEM((NBUF, W), I32),
            pltpu.VMEM((NBUF, W, F), F32),
            pltpu.VMEM((NBUF, W, F), F32),
            pltpu.VMEM((NBUF, W), F32),
            pltpu.VMEM((W // 16, 16, 16), F32),
            pltpu.SemaphoreType.DMA((NBUF,)),
            pltpu.SemaphoreType.DMA((NBUF,)),
            pltpu.SemaphoreType.DMA((NBUF,)),
        ],
    )(_edge1_body)
    return k(xl_p, xr_p, att_p, src, dst)


def _combine_body(acc_ref, den_ref, bias_ref, wl2_ref, wr2_ref, sc_ref,
                  lr2_ref):
    rden = 1.0 / (den_ref[...] + 1e-16)
    xl2 = None
    xr2 = None
    for h in range(H):
        piece = acc_ref[h] * rden[h][:, None] + bias_ref[h][None, :]
        piece = jnp.maximum(piece, 0.0)
        cl = jnp.sum(piece * wl2_ref[h][None, :], axis=1)
        cr = jnp.sum(piece * wr2_ref[h][None, :], axis=1)
        xl2 = cl if xl2 is None else xl2 + cl
        xr2 = cr if xr2 is None else xr2 + cr
    lr2_ref[0, :] = xl2 + sc_ref[0, 0]
    lr2_ref[1, :] = xr2 + sc_ref[0, 1]


def _combine(accs, dens, bias1hf, wl2hf, wr2hf, sc2):
    return pl.pallas_call(
        _combine_body,
        grid=(NB,),
        in_specs=[
            pl.BlockSpec((H, RB, F), lambda i: (0, i, 0)),
            pl.BlockSpec((H, RB), lambda i: (0, i)),
            pl.BlockSpec((H, F), lambda i: (0, 0)),
            pl.BlockSpec((H, F), lambda i: (0, 0)),
            pl.BlockSpec((H, F), lambda i: (0, 0)),
            pl.BlockSpec((1, 8), lambda i: (0, 0)),
        ],
        out_specs=pl.BlockSpec((NC, RB), lambda i: (0, i)),
        out_shape=jax.ShapeDtypeStruct((NC, NP), F32),
    )(accs, dens, bias1hf, wl2hf, wr2hf, sc2)


def _edge2_body(xl2_hbm, xr2_hbm, att_hbm, src_hbm, dst_hbm,
                num_out, den_out, ex_out,
                shnum, shden, xl2_v, xr2_v, att_v, sidx_v, didx_v,
                ex_v, nm_v, zden_v):
    c = lax.axis_index("c")
    s = lax.axis_index("s")
    wid = c * NS + s
    base = wid * EPT
    row0 = s * RPT

    z16 = jnp.zeros((16,), F32)

    def _zden(r, _):
        zden_v[pl.ds(r * 16, 16)] = z16
        return _

    lax.fori_loop(0, RPT // 16, _zden, None)
    pltpu.sync_copy(zden_v, shnum.at[pl.ds(row0, RPT)])
    pltpu.sync_copy(zden_v, shden.at[pl.ds(row0, RPT)])
    pltpu.sync_copy(xl2_hbm, xl2_v)
    pltpu.sync_copy(xr2_hbm, xr2_v)
    pltpu.sync_copy(att_hbm, att_v)
    plsc.subcore_barrier()

    atts = att_v[pl.ds(0, 16)][0]

    def _window(w, _):
        e0 = base + w * W
        pltpu.sync_copy(src_hbm.at[pl.ds(e0, W)], sidx_v)
        pltpu.sync_copy(dst_hbm.at[pl.ds(e0, W)], didx_v)

        def _group(g, _):
            r0 = g * 16
            sv = sidx_v[pl.ds(r0, 16)]
            dv = didx_v[pl.ds(r0, 16)]
            xls = plsc.load_gather(xl2_v, [sv])
            xrd = plsc.load_gather(xr2_v, [dv])
            z = xls + xrd
            m = jnp.maximum(z, 0.2 * z)
            ex = jnp.exp(atts * m)
            ex_v[pl.ds(r0, 16)] = ex
            nm_v[pl.ds(r0, 16)] = ex * xls
            return _

        lax.fori_loop(0, W // 16, _group, None)

        pltpu.sync_copy(ex_v, ex_out.at[pl.ds(e0, W)])
        pltpu.sync_copy(ex_v, shden.at[didx_v], add=True)
        pltpu.sync_copy(nm_v, shnum.at[didx_v], add=True)
        return _

    lax.fori_loop(0, NWIN, _window, None)
    plsc.subcore_barrier()

    out0 = c * NP + row0
    pltpu.sync_copy(shnum.at[pl.ds(row0, RPT)], num_out.at[pl.ds(out0, RPT)])
    pltpu.sync_copy(shden.at[pl.ds(row0, RPT)], den_out.at[pl.ds(out0, RPT)])


def _edge2(xl2, xr2, att2p, src, dst):
    k = functools.partial(
        pl.kernel,
        out_type=(
            jax.ShapeDtypeStruct((NC * NP,), F32),
            jax.ShapeDtypeStruct((NC * NP,), F32),
            jax.ShapeDtypeStruct((E,), F32),
        ),
        mesh=plsc.VectorSubcoreMesh(**_MESH),
        compiler_params=pltpu.CompilerParams(needs_layout_passes=False, use_tc_tiling_on_sc=False),
        scratch_types=[
            pltpu.VMEM_SHARED((NP,), F32),
            pltpu.VMEM_SHARED((NP,), F32),
            pltpu.VMEM((NP,), F32),
            pltpu.VMEM((NP,), F32),
            pltpu.VMEM((16,), F32),
            pltpu.VMEM((W,), I32),
            pltpu.VMEM((W,), I32),
            pltpu.VMEM((W,), F32),
            pltpu.VMEM((W,), F32),
            pltpu.VMEM((RPT,), F32),
        ],
    )(_edge2_body)
    return k(xl2, xr2, att2p, src, dst)


def _final2_body(num_ref, den_ref, sc_ref, h2_ref, rden_ref):
    dent = den_ref[0] + den_ref[1]
    numt = num_ref[0] + num_ref[1]
    r = 1.0 / (dent + 1e-16)
    h2_ref[0, :] = numt * r + sc_ref[0, 2]
    rden_ref[0, :] = r


def _final2(num2, den2, sc2):
    return pl.pallas_call(
        _final2_body,
        grid=(NB,),
        in_specs=[
            pl.BlockSpec((NC, RB), lambda i: (0, i)),
            pl.BlockSpec((NC, RB), lambda i: (0, i)),
            pl.BlockSpec((1, 8), lambda i: (0, 0)),
        ],
        out_specs=[
            pl.BlockSpec((1, RB), lambda i: (0, i)),
            pl.BlockSpec((1, RB), lambda i: (0, i)),
        ],
        out_shape=[
            jax.ShapeDtypeStruct((1, NP), F32),
            jax.ShapeDtypeStruct((1, NP), F32),
        ],
    )(num2, den2, sc2)


def _edge2b_body(rden_hbm, ex_hbm, dst_hbm, a2_out,
                 rden_v, didx_v, ex_v, a_v):
    c = lax.axis_index("c")
    s = lax.axis_index("s")
    base = (c * NS + s) * EPT
    pltpu.sync_copy(rden_hbm, rden_v)

    def _window(w, _):
        e0 = base + w * W
        pltpu.sync_copy(dst_hbm.at[pl.ds(e0, W)], didx_v)
        pltpu.sync_copy(ex_hbm.at[pl.ds(e0, W)], ex_v)

        def _group(g, _):
            r0 = g * 16
            dv = didx_v[pl.ds(r0, 16)]
            r = plsc.load_gather(rden_v, [dv])
            a_v[pl.ds(r0, 16)] = ex_v[pl.ds(r0, 16)] * r
            return _

        lax.fori_loop(0, W // 16, _group, None)
        pltpu.sync_copy(a_v, a2_out.at[pl.ds(e0, W)])
        return _

    lax.fori_loop(0, NWIN, _window, None)


def _edge2b(rden, ex2, dst):
    k = functools.partial(
        pl.kernel,
        out_type=jax.ShapeDtypeStruct((E,), F32),
        mesh=plsc.VectorSubcoreMesh(**_MESH),
        compiler_params=pltpu.CompilerParams(needs_layout_passes=False, use_tc_tiling_on_sc=False),
        scratch_types=[
            pltpu.VMEM((NP,), F32),
            pltpu.VMEM((W,), I32),
            pltpu.VMEM((W,), F32),
            pltpu.VMEM((W,), F32),
        ],
    )(_edge2b_body)
    return k(rden, ex2, dst)


def kernel(x, edge_index, Wl1, bl1, Wr1, br1, att1, bias1,
           Wl2, bl2, Wr2, br2, att2, bias2):
    xp = jnp.pad(x, ((0, NP - N), (0, 0)))
    src = edge_index[0]
    dst = edge_index[1]

    XL, XR = _proj(xp, Wl1, bl1.reshape(1, D_MID), Wr1, br1.reshape(1, D_MID))

    XLp = XL.reshape(HP, NC * NP, F)
    XRp = XR.reshape(HP, NC * NP, F)
    att1p = att1.reshape(HP, 2 * F)
    accs = []
    dens = []
    for p in range(HP):
        acc_p, den_p = _edge1(XLp[p], XRp[p], att1p[p], src, dst)
        accs.append(acc_p.reshape(NC, NP, F))
        dens.append(den_p.reshape(NC, NP))
    accs = jnp.concatenate(accs)
    dens = jnp.concatenate(dens)

    sc2 = jnp.stack([bl2[0], br2[0], bias2[0], att2[0, 0],
                     0.0, 0.0, 0.0, 0.0]).reshape(1, 8).astype(F32)
    att2p = jnp.pad(att2.reshape(-1), (0, 15)).astype(F32)

    lr2 = _combine(accs, dens, bias1.reshape(H, F),
                   Wl2.reshape(H, F), Wr2.reshape(H, F), sc2)

    num2f, den2f, ex2 = _edge2(lr2[0], lr2[1], att2p, src, dst)
    num2 = num2f.reshape(NC, NP)
    den2 = den2f.reshape(NC, NP)

    h2row, rdenrow = _final2(num2, den2, sc2)
    a2 = _edge2b(rdenrow.reshape(NP), ex2, dst)

    h2 = h2row[0, :N].reshape(N, 1)
    return (h2, edge_index, a2.reshape(E, 1))
